# baseline (speedup 1.0000x reference)
import functools

import jax
import jax.numpy as jnp
from jax import lax
from jax.experimental import pallas as pl
from jax.experimental.pallas import tpu as pltpu
from jax.experimental.pallas import tpu_sc as plsc

_K = 17
_RA = 256
_RC = 512
_CH = 128
_NW = 32


def _stage_a(xt_ref, xf_ref, w1_ref, b1_ref, w2_ref, b2_ref,
             tabf_ref, idx_ref):
    b = pl.program_id(0)
    n = xf_ref.shape[1]
    ra = xt_ref.shape[1]
    xt = xt_ref[0]
    xT = xf_ref[0].T
    h = jnp.maximum(
        jnp.dot(xt, w1_ref[...], preferred_element_type=jnp.float32)
        + b1_ref[...], 0.0)
    f = (jnp.dot(h, w2_ref[...], preferred_element_type=jnp.float32)
         + b2_ref[...])
    tabf_ref[0, :, 0:128] = f
    tabf_ref[0, :, 128:131] = xt
    tabf_ref[0, :, 131:256] = jnp.zeros((ra, 125), jnp.float32)

    x2r = jnp.sum(xt * xt, axis=1, keepdims=True)
    x2c = jnp.sum(xT * xT, axis=0, keepdims=True)
    d = (x2r + x2c
         - 2.0 * jnp.dot(xt, xT, preferred_element_type=jnp.float32))

    iota = lax.broadcasted_iota(jnp.int32, (ra, n), 1)
    dbits = lax.bitcast_convert_type(
        jnp.maximum(d, jnp.float32(1e-35)), jnp.int32)
    key = lax.bitcast_convert_type((dbits & jnp.int32(~4095)) | iota,
                                   jnp.float32)
    inf = jnp.float32(jnp.inf)
    m = jnp.min(key, axis=1, keepdims=True)
    sels = []
    for k in range(_K):
        sels.append(m)
        if k + 1 < _K:
            m = jnp.min(jnp.where(key > m, key, inf), axis=1, keepdims=True)
    cols = lax.bitcast_convert_type(jnp.concatenate(sels, axis=1),
                                    jnp.int32) & jnp.int32(4095)
    idx_ref[0] = cols + b * n


def _gather_sc(tab, idx3):
    n_ch = idx3.shape[1]
    per_w = n_ch * _CH
    m = _NW * per_w
    mesh = plsc.VectorSubcoreMesh(core_axis_name="c", subcore_axis_name="s")

    n_pairs = n_ch // 2

    @functools.partial(
        pl.kernel, mesh=mesh,
        out_type=jax.ShapeDtypeStruct((m, 256), jnp.float32),
        scratch_types=[
            pltpu.VMEM((n_ch, _CH), jnp.int32),
            pltpu.VMEM((2, _CH, 256), jnp.float32),
            pltpu.SemaphoreType.DMA,
            pltpu.SemaphoreType.DMA,
        ],
    )
    def gather(tab_hbm, idx_hbm, out_hbm, idx_v, rows_v, sg, so):
        wid = lax.axis_index("s") * 2 + lax.axis_index("c")
        base = wid * per_w
        pltpu.sync_copy(idx_hbm.at[wid], idx_v)
        pltpu.async_copy(tab_hbm.at[idx_v.at[0]], rows_v.at[0], sg)

        def body(p, carry):
            c0 = p * 2
            c1 = c0 + 1
            pltpu.make_async_copy(tab_hbm.at[idx_v.at[c0]], rows_v.at[0],
                                  sg).wait()

            @pl.when(p > 0)
            def _():
                pltpu.make_async_copy(
                    rows_v.at[1],
                    out_hbm.at[pl.ds(base + (c1 - 2) * _CH, _CH)], so).wait()

            pltpu.async_copy(tab_hbm.at[idx_v.at[c1]], rows_v.at[1], sg)
            pltpu.async_copy(rows_v.at[0],
                             out_hbm.at[pl.ds(base + c0 * _CH, _CH)], so)
            pltpu.make_async_copy(tab_hbm.at[idx_v.at[c1]], rows_v.at[1],
                                  sg).wait()
            pltpu.make_async_copy(
                rows_v.at[0],
                out_hbm.at[pl.ds(base + c0 * _CH, _CH)], so).wait()

            @pl.when(p + 1 < n_pairs)
            def _():
                pltpu.async_copy(tab_hbm.at[idx_v.at[c0 + 2]], rows_v.at[0],
                                 sg)

            pltpu.async_copy(rows_v.at[1],
                             out_hbm.at[pl.ds(base + c1 * _CH, _CH)], so)
            return carry

        lax.fori_loop(0, n_pairs, body, 0)
        pltpu.make_async_copy(
            rows_v.at[1],
            out_hbm.at[pl.ds(base + (n_ch - 1) * _CH, _CH)], so).wait()

    return gather(tab, idx3)


def _leaky(x):
    return jnp.where(x >= 0, x, 0.01 * x)


def _stage_c(g_ref, xp_ref, uw_ref, vw_ref, bc1_ref, w2c_ref,
             bc2_ref, w3c_ref, bc3_ref, wkl_ref, wkr_ref, bk_ref,
             wql_ref, wqr_ref, bq_ref, out_ref):
    rc = xp_ref.shape[1]
    g = g_ref[0]
    gfeat = g[:, :, 0:128]
    gx = g[:, :, 128:132]
    xi = xp_ref[0]
    fj = gfeat.reshape(_K * rc, 128)
    xj = gx.reshape(_K * rc, 4)

    u = (jnp.dot(xi, uw_ref[...], preferred_element_type=jnp.float32)
         + bc1_ref[...])
    v = jnp.dot(xj, vw_ref[...], preferred_element_type=jnp.float32)
    h1 = _leaky((v.reshape(_K, rc, 64) + u[None, :, :])).reshape(_K * rc, 64)
    h2 = _leaky(jnp.dot(h1, w2c_ref[...], preferred_element_type=jnp.float32)
                + bc2_ref[...])
    r2 = (jnp.dot(h2, w3c_ref[...], preferred_element_type=jnp.float32)
          + bc3_ref[...])

    kf = (jnp.dot(fj, wkl_ref[...], preferred_element_type=jnp.float32)
          + jnp.dot(r2, wkr_ref[...], preferred_element_type=jnp.float32)
          + bk_ref[...])
    f0 = g[0, :, 0:128]
    r20 = r2.reshape(_K, rc, 128)[0]
    q = (jnp.dot(f0, wql_ref[...], preferred_element_type=jnp.float32)
         + jnp.dot(r20, wqr_ref[...], preferred_element_type=jnp.float32)
         + bq_ref[...])

    lg = jnp.sum(kf.reshape(_K, rc, 256) * q[None, :, :], axis=2)
    kidx = lax.broadcasted_iota(jnp.int32, (_K, rc), 0)
    lg = jnp.where(kidx == 0, jnp.float32(-1e30), lg)
    mx = jnp.max(lg, axis=0, keepdims=True)
    e = jnp.exp(lg - mx)
    w = e / jnp.sum(e, axis=0, keepdims=True)
    o = jnp.sum(w[:, :, None] * gx, axis=0)
    out_ref[0] = o[:, 0:3]


def _full(shape):
    nd = len(shape)
    return pl.BlockSpec(shape, lambda b, i: (0,) * nd)


def kernel(x, global_feat, W1, b1, W2, b2, Wc1, bc1, Wc2, bc2, Wc3, bc3,
           Wq, bq, Wk, bk):
    del global_feat
    B, N, _ = x.shape
    f32 = jnp.float32

    w1t = W1.T
    w2t = W2.T
    b1r, b2r = b1[None, :], b2[None, :]

    tabf, idx = pl.pallas_call(
        _stage_a,
        grid=(B, N // _RA),
        in_specs=[
            pl.BlockSpec((1, _RA, 3), lambda b, i: (b, i, 0)),
            pl.BlockSpec((1, N, 3), lambda b, i: (b, 0, 0)),
            _full((3, 64)), _full((1, 64)),
            _full((64, 128)), _full((1, 128)),
        ],
        out_specs=[
            pl.BlockSpec((1, _RA, 256), lambda b, i: (b, i, 0)),
            pl.BlockSpec((1, _RA, _K), lambda b, i: (b, i, 0)),
        ],
        out_shape=[
            jax.ShapeDtypeStruct((B, N, 256), f32),
            jax.ShapeDtypeStruct((B, N, _K), jnp.int32),
        ],
        compiler_params=pltpu.CompilerParams(
            dimension_semantics=("parallel", "arbitrary")),
    )(x, x, w1t, b1r, w2t, b2r)

    idx3 = jnp.transpose(idx, (0, 2, 1)).reshape(_NW, -1, _CH)
    g = _gather_sc(tabf.reshape(B * N, 256), idx3)
    g = g.reshape(B, _K, N, 256)

    A3, B3, C3 = Wc1[:, 0:3], Wc1[:, 3:6], Wc1[:, 6:9]
    uw = (A3 + C3).T
    vw = jnp.pad((B3 - C3).T, ((0, 1), (0, 0)))
    bc1r, bc2r, bc3r = bc1[None, :], bc2[None, :], bc3[None, :]
    w2c, w3c = Wc2.T, Wc3.T
    wkl, wkr = Wk[:, 0:128].T, Wk[:, 128:256].T
    wql, wqr = Wq[:, 0:128].T, Wq[:, 128:256].T
    bkr, bqr = bk[None, :], bq[None, :]

    out = pl.pallas_call(
        _stage_c,
        grid=(B, N // _RC),
        in_specs=[
            pl.BlockSpec((1, _K, _RC, 256), lambda b, i: (b, 0, i, 0)),
            pl.BlockSpec((1, _RC, 3), lambda b, i: (b, i, 0)),
            _full((3, 64)), _full((4, 64)), _full((1, 64)),
            _full((64, 64)), _full((1, 64)),
            _full((64, 128)), _full((1, 128)),
            _full((128, 256)), _full((128, 256)), _full((1, 256)),
            _full((128, 256)), _full((128, 256)), _full((1, 256)),
        ],
        out_specs=pl.BlockSpec((1, _RC, 3), lambda b, i: (b, i, 0)),
        out_shape=jax.ShapeDtypeStruct((B, N, 3), f32),
        compiler_params=pltpu.CompilerParams(
            dimension_semantics=("parallel", "arbitrary")),
    )(g, x, uw, vw, bc1r, w2c, bc2r, w3c, bc3r,
      wkl, wkr, bkr, wql, wqr, bqr)
    return out


# speedup vs baseline: 1.1581x; 1.1581x over previous
import functools

import jax
import jax.numpy as jnp
from jax import lax
from jax.experimental import pallas as pl
from jax.experimental.pallas import tpu as pltpu
from jax.experimental.pallas import tpu_sc as plsc

_K = 17
_RA = 256
_RC = 512
_CH = 128
_NW = 32


def _stage_a(xt_ref, xf_ref, w1_ref, b1_ref, w2_ref, b2_ref,
             tabf_ref, idx_ref):
    b = pl.program_id(0)
    n = xf_ref.shape[1]
    ra = xt_ref.shape[1]
    xt = xt_ref[0]
    xT = xf_ref[0].T
    h = jnp.maximum(
        jnp.dot(xt, w1_ref[...], preferred_element_type=jnp.float32)
        + b1_ref[...], 0.0)
    f = (jnp.dot(h, w2_ref[...], preferred_element_type=jnp.float32)
         + b2_ref[...])
    tabf_ref[0, :, 0:128] = f
    tabf_ref[0, :, 128:131] = xt
    tabf_ref[0, :, 131:256] = jnp.zeros((ra, 125), jnp.float32)

    x2r = jnp.sum(xt * xt, axis=1, keepdims=True)
    x2c = jnp.sum(xT * xT, axis=0, keepdims=True)
    d = (x2r + x2c
         - 2.0 * jnp.dot(xt, xT, preferred_element_type=jnp.float32))

    iota = lax.broadcasted_iota(jnp.int32, (ra, n), 1)
    dbits = lax.bitcast_convert_type(
        jnp.maximum(d, jnp.float32(1e-35)), jnp.int32)
    key = lax.bitcast_convert_type((dbits & jnp.int32(~4095)) | iota,
                                   jnp.float32)
    inf = jnp.float32(jnp.inf)
    m = jnp.min(key, axis=1, keepdims=True)
    sels = []
    for k in range(_K):
        sels.append(m)
        if k + 1 < _K:
            m = jnp.min(jnp.where(key > m, key, inf), axis=1, keepdims=True)
    cols = lax.bitcast_convert_type(jnp.concatenate(sels, axis=1),
                                    jnp.int32) & jnp.int32(4095)
    idx_ref[0] = cols + b * n


def _gather_sc(tab, idx3):
    n_ch = idx3.shape[1]
    per_w = n_ch * _CH
    m = _NW * per_w
    mesh = plsc.VectorSubcoreMesh(core_axis_name="c", subcore_axis_name="s")

    n_pairs = n_ch // 2

    @functools.partial(
        pl.kernel, mesh=mesh,
        out_type=jax.ShapeDtypeStruct((m, 256), jnp.float32),
        scratch_types=[
            pltpu.VMEM((n_ch, _CH), jnp.int32),
            pltpu.VMEM((2, _CH, 256), jnp.float32),
            pltpu.SemaphoreType.DMA,
            pltpu.SemaphoreType.DMA,
        ],
    )
    def gather(tab_hbm, idx_hbm, out_hbm, idx_v, rows_v, sg, so):
        wid = lax.axis_index("s") * 2 + lax.axis_index("c")
        base = wid * per_w
        pltpu.sync_copy(idx_hbm.at[wid], idx_v)
        pltpu.async_copy(tab_hbm.at[idx_v.at[0]], rows_v.at[0], sg)

        def body(p, carry):
            c0 = p * 2
            c1 = c0 + 1
            pltpu.make_async_copy(tab_hbm.at[idx_v.at[c0]], rows_v.at[0],
                                  sg).wait()

            @pl.when(p > 0)
            def _():
                pltpu.make_async_copy(
                    rows_v.at[1],
                    out_hbm.at[pl.ds(base + (c1 - 2) * _CH, _CH)], so).wait()

            pltpu.async_copy(tab_hbm.at[idx_v.at[c1]], rows_v.at[1], sg)
            pltpu.async_copy(rows_v.at[0],
                             out_hbm.at[pl.ds(base + c0 * _CH, _CH)], so)
            pltpu.make_async_copy(tab_hbm.at[idx_v.at[c1]], rows_v.at[1],
                                  sg).wait()
            pltpu.make_async_copy(
                rows_v.at[0],
                out_hbm.at[pl.ds(base + c0 * _CH, _CH)], so).wait()

            @pl.when(c0 + 2 < n_ch)
            def _():
                pltpu.async_copy(tab_hbm.at[idx_v.at[c0 + 2]], rows_v.at[0],
                                 sg)

            pltpu.async_copy(rows_v.at[1],
                             out_hbm.at[pl.ds(base + c1 * _CH, _CH)], so)
            return carry

        lax.fori_loop(0, n_pairs, body, 0)
        if n_ch % 2:
            c = n_ch - 1
            pltpu.make_async_copy(tab_hbm.at[idx_v.at[c]], rows_v.at[0],
                                  sg).wait()
            pltpu.make_async_copy(
                rows_v.at[1],
                out_hbm.at[pl.ds(base + (c - 1) * _CH, _CH)], so).wait()
            pltpu.sync_copy(rows_v.at[0],
                            out_hbm.at[pl.ds(base + c * _CH, _CH)])
        else:
            pltpu.make_async_copy(
                rows_v.at[1],
                out_hbm.at[pl.ds(base + (n_ch - 1) * _CH, _CH)], so).wait()

    return gather(tab, idx3)


def _leaky(x):
    return jnp.where(x >= 0, x, 0.01 * x)


def _stage_c(g_ref, xp_ref, uw_ref, vw_ref, bc1_ref, w2c_ref,
             bc2_ref, w3c_ref, bc3_ref, wkl_ref, wkr_ref, bk_ref,
             wql_ref, wqr_ref, bq_ref, out_ref):
    rc = xp_ref.shape[1]
    g = g_ref[0]
    gfeat = g[:, :, 0:128]
    gx = g[:, :, 128:132]
    xi = xp_ref[0]
    fj = gfeat.reshape(_K * rc, 128)
    xj = gx.reshape(_K * rc, 4)

    u = (jnp.dot(xi, uw_ref[...], preferred_element_type=jnp.float32)
         + bc1_ref[...])
    v = jnp.dot(xj, vw_ref[...], preferred_element_type=jnp.float32)
    h1 = _leaky((v.reshape(_K, rc, 64) + u[None, :, :])).reshape(_K * rc, 64)
    h2 = _leaky(jnp.dot(h1, w2c_ref[...], preferred_element_type=jnp.float32)
                + bc2_ref[...])
    r2 = (jnp.dot(h2, w3c_ref[...], preferred_element_type=jnp.float32)
          + bc3_ref[...])

    kf = (jnp.dot(fj, wkl_ref[...], preferred_element_type=jnp.float32)
          + jnp.dot(r2, wkr_ref[...], preferred_element_type=jnp.float32)
          + bk_ref[...])
    f0 = g[0, :, 0:128]
    r20 = r2.reshape(_K, rc, 128)[0]
    q = (jnp.dot(f0, wql_ref[...], preferred_element_type=jnp.float32)
         + jnp.dot(r20, wqr_ref[...], preferred_element_type=jnp.float32)
         + bq_ref[...])

    lg = jnp.sum(kf.reshape(_K, rc, 256) * q[None, :, :], axis=2)
    kidx = lax.broadcasted_iota(jnp.int32, (_K, rc), 0)
    lg = jnp.where(kidx == 0, jnp.float32(-1e30), lg)
    mx = jnp.max(lg, axis=0, keepdims=True)
    e = jnp.exp(lg - mx)
    w = e / jnp.sum(e, axis=0, keepdims=True)
    o = jnp.sum(w[:, :, None] * gx, axis=0)
    out_ref[0] = o[:, 0:3]


def _full(shape):
    nd = len(shape)
    return pl.BlockSpec(shape, lambda b, i: (0,) * nd)


def kernel(x, global_feat, W1, b1, W2, b2, Wc1, bc1, Wc2, bc2, Wc3, bc3,
           Wq, bq, Wk, bk):
    del global_feat
    B, N, _ = x.shape
    f32 = jnp.float32

    w1t = W1.T
    w2t = W2.T
    b1r, b2r = b1[None, :], b2[None, :]

    A3, B3, C3 = Wc1[:, 0:3], Wc1[:, 3:6], Wc1[:, 6:9]
    uw = (A3 + C3).T
    vw = jnp.pad((B3 - C3).T, ((0, 1), (0, 0)))
    bc1r, bc2r, bc3r = bc1[None, :], bc2[None, :], bc3[None, :]
    w2c, w3c = Wc2.T, Wc3.T
    wkl, wkr = Wk[:, 0:128].T, Wk[:, 128:256].T
    wql, wqr = Wq[:, 0:128].T, Wq[:, 128:256].T
    bkr, bqr = bk[None, :], bq[None, :]

    outs = []
    for b in range(B):
        xb = x[b:b + 1]
        tabb, idxb = pl.pallas_call(
            _stage_a,
            grid=(1, N // _RA),
            in_specs=[
                pl.BlockSpec((1, _RA, 3), lambda b, i: (b, i, 0)),
                pl.BlockSpec((1, N, 3), lambda b, i: (b, 0, 0)),
                _full((3, 64)), _full((1, 64)),
                _full((64, 128)), _full((1, 128)),
            ],
            out_specs=[
                pl.BlockSpec((1, _RA, 256), lambda b, i: (b, i, 0)),
                pl.BlockSpec((1, _RA, _K), lambda b, i: (b, i, 0)),
            ],
            out_shape=[
                jax.ShapeDtypeStruct((1, N, 256), f32),
                jax.ShapeDtypeStruct((1, N, _K), jnp.int32),
            ],
            compiler_params=pltpu.CompilerParams(
                dimension_semantics=("parallel", "arbitrary")),
        )(xb, xb, w1t, b1r, w2t, b2r)

        idx3b = jnp.transpose(idxb, (0, 2, 1)).reshape(_NW, -1, _CH)
        gb = _gather_sc(tabb.reshape(N, 256), idx3b)
        gb = gb.reshape(1, _K, N, 256)

        outs.append(pl.pallas_call(
            _stage_c,
            grid=(1, N // _RC),
            in_specs=[
                pl.BlockSpec((1, _K, _RC, 256), lambda b, i: (b, 0, i, 0)),
                pl.BlockSpec((1, _RC, 3), lambda b, i: (b, i, 0)),
                _full((3, 64)), _full((4, 64)), _full((1, 64)),
                _full((64, 64)), _full((1, 64)),
                _full((64, 128)), _full((1, 128)),
                _full((128, 256)), _full((128, 256)), _full((1, 256)),
                _full((128, 256)), _full((128, 256)), _full((1, 256)),
            ],
            out_specs=pl.BlockSpec((1, _RC, 3), lambda b, i: (b, i, 0)),
            out_shape=jax.ShapeDtypeStruct((1, N, 3), f32),
            compiler_params=pltpu.CompilerParams(
                dimension_semantics=("parallel", "arbitrary")),
        )(gb, xb, uw, vw, bc1r, w2c, bc2r, w3c, bc3r,
          wkl, wkr, bkr, wql, wqr, bqr))
    return jnp.concatenate(outs, axis=0)
